# baseline (speedup 1.0000x reference)
import functools

import jax
import jax.numpy as jnp
from jax import lax
from jax.experimental import pallas as pl
from jax.experimental.pallas import tpu as pltpu
from jax.experimental.pallas import tpu_sc as plsc

_M = 16384
_A = 200
_NC = 2
_NS = 16
_NW = _NC * _NS
_L = 16
_MPW = _M // _NW


def _tec_body(species_hbm, energies_hbm, table_hbm, out_hbm,
              spc_v, eng_v, out_v, tbl_v):
    wid = lax.axis_index("s") * _NC + lax.axis_index("c")
    base = wid * _MPW

    pltpu.sync_copy(table_hbm, tbl_v)
    pltpu.sync_copy(energies_hbm.at[pl.ds(base, _MPW)], eng_v)
    pltpu.sync_copy(species_hbm.at[pl.ds(base * _A, _MPW * _A)], spc_v)

    tbl = tbl_v[...]
    lanes = lax.iota(jnp.int32, _L)

    def group_body(g, carry):
        idx0 = g * (_L * _A) + lanes * _A

        def atom_body(a, acc):
            sp = plsc.load_gather(spc_v, [idx0 + a])
            e = tbl.at[sp].get(mode="promise_in_bounds")
            return acc + e

        acc = lax.fori_loop(0, _A, atom_body, jnp.zeros((_L,), jnp.float32))
        out_v[pl.ds(g * _L, _L)] = acc + eng_v[pl.ds(g * _L, _L)]
        return carry

    lax.fori_loop(0, _MPW // _L, group_body, 0)
    pltpu.sync_copy(out_v, out_hbm.at[pl.ds(base, _MPW)])


_shift = functools.partial(
    pl.kernel,
    out_type=jax.ShapeDtypeStruct((_M,), jnp.float32),
    mesh=plsc.VectorSubcoreMesh(core_axis_name="c", subcore_axis_name="s"),
    scratch_types=[
        pltpu.VMEM((_MPW * _A,), jnp.int32),
        pltpu.VMEM((_MPW,), jnp.float32),
        pltpu.VMEM((_MPW,), jnp.float32),
        pltpu.VMEM((_L,), jnp.float32),
    ],
    compiler_params=pltpu.CompilerParams(needs_layout_passes=False),
)(_tec_body)


@jax.jit
def kernel(species, energies, self_energies):
    table = jnp.pad(self_energies.astype(jnp.float32), (0, _L - self_energies.shape[0]))
    shifted = _shift(species.reshape(-1), energies, table)
    return species, shifted


# speedup vs baseline: 1.2970x; 1.2970x over previous
import functools

import jax
import jax.numpy as jnp
from jax import lax
from jax.experimental import pallas as pl
from jax.experimental.pallas import tpu as pltpu
from jax.experimental.pallas import tpu_sc as plsc

_M = 16384
_A = 200
_NC = 2
_NS = 16
_NW = _NC * _NS
_L = 16
_MPW = _M // _NW


def _tec_body(species_hbm, energies_hbm, table_hbm, out_hbm,
              spc_v, eng_v, out_v, tbl_v):
    wid = lax.axis_index("s") * _NC + lax.axis_index("c")
    base = wid * _MPW

    pltpu.sync_copy(table_hbm, tbl_v)
    pltpu.sync_copy(energies_hbm.at[pl.ds(base, _MPW)], eng_v)
    pltpu.sync_copy(species_hbm.at[pl.ds(base * _A, _MPW * _A)], spc_v)

    tbl = tbl_v[...]
    lanes = lax.iota(jnp.int32, _L)

    def group_body(g, carry):
        idx0 = g * (_L * _A) + lanes * _A

        def atom_body(i, accs):
            a0 = i * 8
            acc0, acc1 = accs
            e = []
            for j in range(8):
                sp = plsc.load_gather(spc_v, [idx0 + (a0 + j)])
                e.append(tbl.at[sp].get(mode="promise_in_bounds"))
            s0 = (e[0] + e[1]) + (e[2] + e[3])
            s1 = (e[4] + e[5]) + (e[6] + e[7])
            return (acc0 + s0, acc1 + s1)

        z = jnp.zeros((_L,), jnp.float32)
        acc0, acc1 = lax.fori_loop(0, _A // 8, atom_body, (z, z), unroll=5)
        out_v[pl.ds(g * _L, _L)] = (acc0 + acc1) + eng_v[pl.ds(g * _L, _L)]
        return carry

    lax.fori_loop(0, _MPW // _L, group_body, 0)
    pltpu.sync_copy(out_v, out_hbm.at[pl.ds(base, _MPW)])


_shift = functools.partial(
    pl.kernel,
    out_type=jax.ShapeDtypeStruct((_M,), jnp.float32),
    mesh=plsc.VectorSubcoreMesh(core_axis_name="c", subcore_axis_name="s"),
    scratch_types=[
        pltpu.VMEM((_MPW * _A,), jnp.int32),
        pltpu.VMEM((_MPW,), jnp.float32),
        pltpu.VMEM((_MPW,), jnp.float32),
        pltpu.VMEM((_L,), jnp.float32),
    ],
    compiler_params=pltpu.CompilerParams(needs_layout_passes=False),
)(_tec_body)


@jax.jit
def kernel(species, energies, self_energies):
    table = jnp.pad(self_energies.astype(jnp.float32), (0, _L - self_energies.shape[0]))
    shifted = _shift(species.reshape(-1), energies, table)
    return species, shifted
